# Initial kernel scaffold; baseline (speedup 1.0000x reference)
import jax
import jax.numpy as jnp
from jax.experimental import pallas as pl


def kernel(x, pos_table):
    raise NotImplementedError("write your pallas kernel here")


# speedup vs baseline: 1.2163x; 1.2163x over previous
import jax
import jax.numpy as jnp
from jax.experimental import pallas as pl


def _add_block(x_ref, pos_ref, o_ref):
    o_ref[...] = x_ref[...] + pos_ref[...]


def kernel(x, pos_table):
    B, S, N = x.shape
    BS = 512
    grid = (B, S // BS)
    return pl.pallas_call(
        _add_block,
        grid=grid,
        in_specs=[
            pl.BlockSpec((1, BS, N), lambda b, s: (b, s, 0)),
            pl.BlockSpec((1, BS, N), lambda b, s: (0, s, 0)),
        ],
        out_specs=pl.BlockSpec((1, BS, N), lambda b, s: (b, s, 0)),
        out_shape=jax.ShapeDtypeStruct((B, S, N), x.dtype),
    )(x, pos_table[None, :S, :])


# speedup vs baseline: 1.4911x; 1.2259x over previous
import jax
import jax.numpy as jnp
from jax.experimental import pallas as pl


def _add_block(x_ref, pos_ref, o_ref):
    o_ref[...] = x_ref[...] + pos_ref[...]


def kernel(x, pos_table):
    B, S, N = x.shape
    BS = 1024
    grid = (S // BS, B)
    return pl.pallas_call(
        _add_block,
        grid=grid,
        in_specs=[
            pl.BlockSpec((1, BS, N), lambda s, b: (b, s, 0)),
            pl.BlockSpec((1, BS, N), lambda s, b: (0, s, 0)),
        ],
        out_specs=pl.BlockSpec((1, BS, N), lambda s, b: (b, s, 0)),
        out_shape=jax.ShapeDtypeStruct((B, S, N), x.dtype),
    )(x, pos_table[None, :S, :])


# speedup vs baseline: 1.5750x; 1.0563x over previous
import jax
import jax.numpy as jnp
from jax.experimental import pallas as pl


def _add_block(x_ref, pos_ref, o_ref):
    o_ref[...] = x_ref[...] + pos_ref[...]


def kernel(x, pos_table):
    B, S, N = x.shape
    BS = 2048
    grid = (S // BS, B)
    return pl.pallas_call(
        _add_block,
        grid=grid,
        in_specs=[
            pl.BlockSpec((1, BS, N), lambda s, b: (b, s, 0)),
            pl.BlockSpec((1, BS, N), lambda s, b: (0, s, 0)),
        ],
        out_specs=pl.BlockSpec((1, BS, N), lambda s, b: (b, s, 0)),
        out_shape=jax.ShapeDtypeStruct((B, S, N), x.dtype),
    )(x, pos_table[None, :S, :])
